# baseline (speedup 1.0000x reference)
import functools

import jax
import jax.numpy as jnp
import numpy as np
from jax import lax
from jax.experimental import pallas as pl
from jax.experimental.pallas import tpu as pltpu
from jax.experimental.pallas import tpu_sc as plsc

N_NODES = 10000
N_EDGES = 160000
D = 80
NS = 32
NV = 48

_NC = 2
_NSUB = 16
_NW = _NC * _NSUB
DP = 128
NP = 10240
EP = 163840
_CG = 128
_KG = EP // (_CG * _NW)
_CS = 500
_KS = N_EDGES // (_CS * _NW)
_RPT = N_NODES // _NSUB


@functools.lru_cache(maxsize=None)
def _get_gather2():
    mesh = plsc.VectorSubcoreMesh(core_axis_name="c", subcore_axis_name="s")

    @functools.partial(
        pl.kernel, mesh=mesh,
        out_type=(jax.ShapeDtypeStruct((EP, DP), jnp.float32),
                  jax.ShapeDtypeStruct((EP, DP), jnp.float32)),
        scratch_types=[pltpu.VMEM((_KG, _CG), jnp.int32),
                       pltpu.VMEM((_CG, DP), jnp.float32),
                       pltpu.VMEM((_CG, DP), jnp.float32),
                       pltpu.SemaphoreType.DMA,
                       pltpu.SemaphoreType.DMA],
        compiler_params=pltpu.CompilerParams(use_tc_tiling_on_sc=True),
    )
    def sc_gather2(tab_a, idx_a, tab_b, idx_b, out_a, out_b,
                   idx_all, r0, r1, s0, s1):
        wid = lax.axis_index("s") * _NC + lax.axis_index("c")
        rbufs, sems = (r0, r1), (s0, s1)
        for tab, idx2, out in ((tab_a, idx_a, out_a), (tab_b, idx_b, out_b)):
            j0 = wid * _KG
            pltpu.sync_copy(idx2.at[pl.ds(j0, _KG)], idx_all)
            cp = pltpu.async_copy(tab.at[idx_all.at[0]], rbufs[0], sems[0])
            for k in range(_KG):
                nb = (k + 1) % 2
                cpn = None
                if k + 1 < _KG:
                    cpn = pltpu.async_copy(tab.at[idx_all.at[k + 1]],
                                           rbufs[nb], sems[nb])
                cp.wait()
                pltpu.sync_copy(rbufs[k % 2],
                                out.at[pl.ds((j0 + k) * _CG, _CG)])
                cp = cpn

    return sc_gather2


@functools.lru_cache(maxsize=None)
def _get_scatter2():
    mesh = plsc.VectorSubcoreMesh(core_axis_name="c", subcore_axis_name="s")

    @functools.partial(
        pl.kernel, mesh=mesh,
        out_type=(jax.ShapeDtypeStruct((_NC * N_NODES, D), jnp.float32),
                  jax.ShapeDtypeStruct((_NC * N_NODES, D), jnp.float32)),
        scratch_types=[pltpu.VMEM((_CS,), jnp.int32),
                       pltpu.VMEM((_CS,), jnp.int32),
                       pltpu.VMEM((_CS, D), jnp.float32),
                       pltpu.VMEM((_CS, D), jnp.float32),
                       pltpu.VMEM_SHARED((N_NODES, D), jnp.float32),
                       pltpu.SemaphoreType.DMA,
                       pltpu.SemaphoreType.DMA],
        compiler_params=pltpu.CompilerParams(use_tc_tiling_on_sc=False),
    )
    def sc_scatter2(msg_l, idx_r, msg_r, idx_l, zeros_blk, agg_l, agg_r,
                    i0, i1, m0, m1, acc, s0, s1):
        cid = lax.axis_index("c")
        sid = lax.axis_index("s")
        wid = sid * _NC + cid
        r0 = sid * _RPT
        ibufs, mbufs, sems = (i0, i1), (m0, m1), (s0, s1)
        for msg, idx2, agg in ((msg_l, idx_r, agg_l), (msg_r, idx_l, agg_r)):
            pltpu.sync_copy(zeros_blk, acc.at[pl.ds(r0, _RPT)])
            plsc.subcore_barrier()
            j0 = wid * _KS
            pltpu.sync_copy(idx2.at[j0], ibufs[0])
            cp = pltpu.async_copy(msg.at[pl.ds(j0 * _CS, _CS)], mbufs[0],
                                  sems[0])
            for k in range(_KS):
                nb = (k + 1) % 2
                cpn = None
                if k + 1 < _KS:
                    pltpu.sync_copy(idx2.at[j0 + k + 1], ibufs[nb])
                    cpn = pltpu.async_copy(
                        msg.at[pl.ds((j0 + k + 1) * _CS, _CS)], mbufs[nb],
                        sems[nb])
                cp.wait()
                pltpu.sync_copy(mbufs[k % 2], acc.at[ibufs[k % 2]], add=True)
                cp = cpn
            plsc.subcore_barrier()
            pltpu.sync_copy(acc.at[pl.ds(r0, _RPT)],
                            agg.at[pl.ds(cid * N_NODES + r0, _RPT)])
            plsc.subcore_barrier()

    return sc_scatter2


_BF = jnp.bfloat16


def _dot(a, b):
    return jnp.dot(a, b, preferred_element_type=jnp.float32)


_WNAMES = ['sfw1', 'sfb1', 'sfw2', 'sfb2', 'w_bpnq_s', 'b_bpnq', 'w_bpnq_v',
           'w_f1s', 'b_f1', 'w_f1v', 's32', 's32t', 'w_f2s', 'b_f2', 'w_f2v',
           'w_g1', 'b_g1', 'w_g2', 'b_g2', 'wns', 'bns', 'wnv']


def _dense_body(e_in, esc, eattr, hl, hr, *wrefs, msgl, msgr, part):
    w = {k: wrefs[i][...] for i, k in enumerate(_WNAMES)}
    e = e_in[...].astype(_BF)
    v_b = e[:, NS:]
    x = _dot(jnp.concatenate([e[:, :NS], esc[...].astype(_BF)], axis=1),
             w['sfw1']) + w['sfb1']
    x = x * jax.nn.sigmoid(x)
    s_b = (_dot(x.astype(_BF), w['sfw2']) + w['sfb2']).astype(_BF)
    ea = eattr[...].astype(_BF)
    s_sh = ea[:, 0:1]
    v_sh = ea[:, 1:4]
    hlv = hl[...][:, :D].astype(_BF)
    hrv = hr[...][:, :D].astype(_BF)
    sblr = jnp.concatenate([s_b, hlv[:, :NS], hrv[:, :NS]], axis=1)
    vblr = jnp.concatenate([v_b, hlv[:, NS:], hrv[:, NS:]], axis=1)
    bpnq_s = _dot(sblr, w['w_bpnq_s']) + w['b_bpnq']
    bpnq_v = _dot(vblr, w['w_bpnq_v'])
    f_s = _dot(jnp.concatenate([bpnq_s.astype(_BF), s_sh], axis=1),
               w['w_f1s']) + w['b_f1']
    f_v = _dot(jnp.concatenate([bpnq_v.astype(_BF), v_sh], axis=1),
               w['w_f1v'])
    f_s = f_s * jax.nn.sigmoid(f_s)
    nrm = jnp.sqrt(_dot((f_v * f_v).astype(_BF), w['s32']))
    scale = jax.nn.sigmoid(nrm) * nrm / (nrm + 1e-8)
    f_v = f_v * _dot(scale.astype(_BF), w['s32t'])
    f_s = _dot(f_s.astype(_BF), w['w_f2s']) + w['b_f2']
    f_v = _dot(f_v.astype(_BF), w['w_f2v'])
    g = _dot(jnp.concatenate([sblr, s_sh], axis=1), w['w_g1']) + w['b_g1']
    g = g * jax.nn.sigmoid(g)
    gates = jax.nn.sigmoid(_dot(g.astype(_BF), w['w_g2']) + w['b_g2'])
    p_s = _dot(sblr, w['wns']) + w['bns']
    p_v = _dot(vblr, w['wnv'])
    msgl[...] = jnp.concatenate(
        [f_s[:, :NS], f_v[:, :NV]], axis=1) * gates[:, 0:1]
    msgr[...] = jnp.concatenate(
        [f_s[:, NS:], f_v[:, NV:]], axis=1) * gates[:, 1:2]
    part[...] = jnp.concatenate([p_s, p_v], axis=1)


_BE = 4000


def _dense_call(e_in, esc, eattr, hl, hr, weights):
    grid = (N_EDGES // _BE,)
    row_spec = lambda width: pl.BlockSpec((_BE, width), lambda i: (i, 0))
    full = lambda a: pl.BlockSpec(a.shape, lambda i: (0,) * a.ndim)
    in_specs = [row_spec(D), row_spec(16), row_spec(4), row_spec(DP),
                row_spec(DP)]
    in_specs += [full(a) for a in weights]
    out_specs = [row_spec(D)] * 3
    out_shape = [jax.ShapeDtypeStruct((N_EDGES, D), jnp.float32)] * 3

    def body(*refs):
        _dense_body(*refs[:-3], msgl=refs[-3], msgr=refs[-2], part=refs[-1])

    return pl.pallas_call(
        body, grid=grid, in_specs=in_specs, out_specs=out_specs,
        out_shape=out_shape,
        compiler_params=pltpu.CompilerParams(
            dimension_semantics=("arbitrary",)),
    )(e_in, esc, eattr, hl, hr, *weights)


def _combine_body(al, ar, ol, orr):
    zr = jnp.zeros((NP - N_NODES, D), jnp.float32)
    zl = jnp.zeros((NP, DP - D), jnp.float32)
    a = al[...]
    ol[...] = jnp.concatenate(
        [jnp.concatenate([a[:N_NODES] + a[N_NODES:], zr], axis=0), zl], axis=1)
    b = ar[...]
    orr[...] = jnp.concatenate(
        [jnp.concatenate([b[:N_NODES] + b[N_NODES:], zr], axis=0), zl], axis=1)


def _combine_call(agg_l2, agg_r2):
    return pl.pallas_call(
        _combine_body,
        out_shape=[jax.ShapeDtypeStruct((NP, DP), jnp.float32)] * 2,
    )(agg_l2, agg_r2)


_BA = 8000


def _final_add_body(p, fl, fr, o):
    o[...] = p[...] + fl[...][:, :D] + fr[...][:, :D]


def _final_add_call(part, fl, fr):
    spec = pl.BlockSpec((_BA, D), lambda i: (i, 0))
    pspec = pl.BlockSpec((_BA, DP), lambda i: (i, 0))
    return pl.pallas_call(
        _final_add_body, grid=(N_EDGES // _BA,),
        in_specs=[spec, pspec, pspec], out_specs=spec,
        out_shape=jax.ShapeDtypeStruct((N_EDGES, D), jnp.float32),
        compiler_params=pltpu.CompilerParams(
            dimension_semantics=("arbitrary",)),
    )(part, fl, fr)


def _kron3(w):
    return jnp.kron(w, jnp.eye(3, dtype=w.dtype))


def _expand_weights(p):
    r32, r16 = 1 / np.sqrt(32), 1 / np.sqrt(16)
    r33, r17 = 1 / np.sqrt(33), 1 / np.sqrt(17)
    r65 = 1 / np.sqrt(65)
    k3 = _kron3
    f32 = jnp.float32
    cat = jnp.concatenate

    w_bpnq_s = (jnp.zeros((96, 64), f32)
                .at[0:32, 0:16].set(p['L_bl_w0'] * r32)
                .at[0:32, 32:48].set(p['R_bl_w0'] * r32)
                .at[32:64, 16:32].set(p['L_nl_w0'] * r32)
                .at[64:96, 48:64].set(p['R_nl_w0'] * r32))
    b_bpnq = cat([p['L_bl_b'], p['L_nl_b'],
                  p['R_bl_b'], p['R_nl_b']])[None, :]
    w_bpnq_v = (jnp.zeros((144, 96), f32)
                .at[0:48, 0:24].set(k3(p['L_bl_w1']) * r16)
                .at[0:48, 48:72].set(k3(p['R_bl_w1']) * r16)
                .at[48:96, 24:48].set(k3(p['L_nl_w1']) * r16)
                .at[96:144, 72:96].set(k3(p['R_nl_w1']) * r16))
    w_f1s = (jnp.zeros((65, 64), f32)
             .at[0:32, 0:32].set(p['L_f1_w0'][0:32] * r33)
             .at[32:64, 32:64].set(p['R_f1_w0'][0:32] * r33)
             .at[64:65, 0:32].set(p['L_f1_w0'][32:33] * r33)
             .at[64:65, 32:64].set(p['R_f1_w0'][32:33] * r33))
    b_f1 = cat([p['L_f1_b'], p['R_f1_b']])[None, :]
    w_f1v = (jnp.zeros((99, 96), f32)
             .at[0:48, 0:48].set(k3(p['L_f1_w1'][0:16]) * r17)
             .at[48:96, 48:96].set(k3(p['R_f1_w1'][0:16]) * r17)
             .at[96:99, 0:48].set(k3(p['L_f1_w1'][16:17]) * r17)
             .at[96:99, 48:96].set(k3(p['R_f1_w1'][16:17]) * r17))
    s32 = jnp.kron(jnp.eye(32, dtype=f32), jnp.ones((3, 1), f32))
    w_f2s = (jnp.zeros((64, 64), f32)
             .at[0:32, 0:32].set(p['L_f2_w0'] * r32)
             .at[32:64, 32:64].set(p['R_f2_w0'] * r32))
    b_f2 = cat([p['L_f2_b'], p['R_f2_b']])[None, :]
    w_f2v = (jnp.zeros((96, 96), f32)
             .at[0:48, 0:48].set(k3(p['L_f2_w1']) * r16)
             .at[48:96, 48:96].set(k3(p['R_f2_w1']) * r16))
    w_g1 = (jnp.zeros((97, 130), f32)
            .at[0:32, 0:65].set(p['L_g1_w0'][0:32] * r65)
            .at[0:32, 65:130].set(p['R_g1_w0'][0:32] * r65)
            .at[32:64, 0:65].set(p['L_g1_w0'][32:64] * r65)
            .at[64:96, 65:130].set(p['R_g1_w0'][32:64] * r65)
            .at[96:97, 0:65].set(p['L_g1_w0'][64:65] * r65)
            .at[96:97, 65:130].set(p['R_g1_w0'][64:65] * r65))
    b_g1 = cat([p['L_g1_b'], p['R_g1_b']])[None, :]
    w_g2 = (jnp.zeros((130, 2), f32)
            .at[0:65, 0:1].set(p['L_g2_w0'] * r65)
            .at[65:130, 1:2].set(p['R_g2_w0'] * r65))
    b_g2 = cat([p['L_g2_b'], p['R_g2_b']])[None, :]
    wns = cat([p['self_w0'], p['nfl_w0'], p['nfr_w0']], axis=0) * r32
    bns = (p['nfl_b'] + p['nfr_b'] + p['self_b'])[None, :]
    wnv = cat([k3(p['self_w1']), k3(p['nfl_w1']), k3(p['nfr_w1'])],
              axis=0) * r16
    d = {
        'sfw1': p['sf_w1'], 'sfb1': p['sf_b1'][None, :],
        'sfw2': p['sf_w2'], 'sfb2': p['sf_b2'][None, :],
        'w_bpnq_s': w_bpnq_s, 'b_bpnq': b_bpnq, 'w_bpnq_v': w_bpnq_v,
        'w_f1s': w_f1s, 'b_f1': b_f1, 'w_f1v': w_f1v,
        's32': s32, 's32t': s32.T,
        'w_f2s': w_f2s, 'b_f2': b_f2, 'w_f2v': w_f2v,
        'w_g1': w_g1, 'b_g1': b_g1, 'w_g2': w_g2, 'b_g2': b_g2,
        'wns': wns, 'bns': bns, 'wnv': wnv,
    }
    out = []
    for k in _WNAMES:
        a = d[k]
        out.append(a if k.startswith('b') or k.startswith('sfb')
                   else a.astype(_BF))
    return out


def kernel(h, e_in, edge_scalars, edge_attr, edge_index, params):
    left = edge_index[0]
    right = edge_index[1]
    lp = jnp.pad(left, (0, EP - N_EDGES))
    rp = jnp.pad(right, (0, EP - N_EDGES))
    lg = lp.reshape(EP // _CG, _CG)
    rg = rp.reshape(EP // _CG, _CG)
    ls = left.reshape(N_EDGES // _CS, _CS)
    rs = right.reshape(N_EDGES // _CS, _CS)
    weights = _expand_weights(params)
    h_pad = jnp.pad(h, ((0, NP - N_NODES), (0, DP - D)))
    hl, hr = _get_gather2()(h_pad, lg, h_pad, rg)
    msgl, msgr, part = _dense_call(e_in, edge_scalars, edge_attr, hl, hr,
                                   weights)
    zeros_blk = jnp.zeros((_RPT, D), jnp.float32)
    agg_l2, agg_r2 = _get_scatter2()(msgl, rs, msgr, ls, zeros_blk)
    agg_l, agg_r = _combine_call(agg_l2, agg_r2)
    fl, fr = _get_gather2()(agg_l, lg, agg_r, rg)
    return _final_add_call(part, fl, fr)


# speedup vs baseline: 1.0758x; 1.0758x over previous
import functools

import jax
import jax.numpy as jnp
import numpy as np
from jax import lax
from jax.experimental import pallas as pl
from jax.experimental.pallas import tpu as pltpu
from jax.experimental.pallas import tpu_sc as plsc

N_NODES = 10000
N_EDGES = 160000
D = 80
NS = 32
NV = 48

_NC = 2
_NSUB = 16
_NW = _NC * _NSUB
DP = 128
NP = 10240
EP = 163840
_CG = 128
_KG = EP // (_CG * _NW)
_CS = 500
_KS = N_EDGES // (_CS * _NW)
_RPT = N_NODES // _NSUB


@functools.lru_cache(maxsize=None)
def _get_gather2():
    mesh = plsc.VectorSubcoreMesh(core_axis_name="c", subcore_axis_name="s")

    @functools.partial(
        pl.kernel, mesh=mesh,
        out_type=(jax.ShapeDtypeStruct((EP, DP), jnp.float32),
                  jax.ShapeDtypeStruct((EP, DP), jnp.float32)),
        scratch_types=[pltpu.VMEM((_KG, _CG), jnp.int32),
                       pltpu.VMEM((_CG, DP), jnp.float32),
                       pltpu.VMEM((_CG, DP), jnp.float32),
                       pltpu.SemaphoreType.DMA,
                       pltpu.SemaphoreType.DMA],
        compiler_params=pltpu.CompilerParams(use_tc_tiling_on_sc=False),
    )
    def sc_gather2(tab_a, idx_a, tab_b, idx_b, out_a, out_b,
                   idx_all, r0, r1, s0, s1):
        wid = lax.axis_index("s") * _NC + lax.axis_index("c")
        rbufs, sems = (r0, r1), (s0, s1)
        for tab, idx2, out in ((tab_a, idx_a, out_a), (tab_b, idx_b, out_b)):
            j0 = wid * _KG
            pltpu.sync_copy(idx2.at[pl.ds(j0, _KG)], idx_all)
            cp = pltpu.async_copy(tab.at[idx_all.at[0]], rbufs[0], sems[0])
            for k in range(_KG):
                nb = (k + 1) % 2
                cpn = None
                if k + 1 < _KG:
                    cpn = pltpu.async_copy(tab.at[idx_all.at[k + 1]],
                                           rbufs[nb], sems[nb])
                cp.wait()
                pltpu.sync_copy(rbufs[k % 2],
                                out.at[pl.ds((j0 + k) * _CG, _CG)])
                cp = cpn

    return sc_gather2


@functools.lru_cache(maxsize=None)
def _get_scatter2():
    mesh = plsc.VectorSubcoreMesh(core_axis_name="c", subcore_axis_name="s")

    @functools.partial(
        pl.kernel, mesh=mesh,
        out_type=(jax.ShapeDtypeStruct((_NC * N_NODES, D), jnp.float32),
                  jax.ShapeDtypeStruct((_NC * N_NODES, D), jnp.float32)),
        scratch_types=[pltpu.VMEM((_CS,), jnp.int32),
                       pltpu.VMEM((_CS,), jnp.int32),
                       pltpu.VMEM((_CS, D), jnp.float32),
                       pltpu.VMEM((_CS, D), jnp.float32),
                       pltpu.VMEM_SHARED((N_NODES, D), jnp.float32),
                       pltpu.SemaphoreType.DMA,
                       pltpu.SemaphoreType.DMA],
        compiler_params=pltpu.CompilerParams(use_tc_tiling_on_sc=False),
    )
    def sc_scatter2(msg_l, idx_r, msg_r, idx_l, zeros_blk, agg_l, agg_r,
                    i0, i1, m0, m1, acc, s0, s1):
        cid = lax.axis_index("c")
        sid = lax.axis_index("s")
        wid = sid * _NC + cid
        r0 = sid * _RPT
        ibufs, mbufs, sems = (i0, i1), (m0, m1), (s0, s1)
        for msg, idx2, agg in ((msg_l, idx_r, agg_l), (msg_r, idx_l, agg_r)):
            pltpu.sync_copy(zeros_blk, acc.at[pl.ds(r0, _RPT)])
            plsc.subcore_barrier()
            j0 = wid * _KS
            pltpu.sync_copy(idx2.at[j0], ibufs[0])
            cp = pltpu.async_copy(msg.at[pl.ds(j0 * _CS, _CS)], mbufs[0],
                                  sems[0])
            for k in range(_KS):
                nb = (k + 1) % 2
                cpn = None
                if k + 1 < _KS:
                    pltpu.sync_copy(idx2.at[j0 + k + 1], ibufs[nb])
                    cpn = pltpu.async_copy(
                        msg.at[pl.ds((j0 + k + 1) * _CS, _CS)], mbufs[nb],
                        sems[nb])
                cp.wait()
                pltpu.sync_copy(mbufs[k % 2], acc.at[ibufs[k % 2]], add=True)
                cp = cpn
            plsc.subcore_barrier()
            pltpu.sync_copy(acc.at[pl.ds(r0, _RPT)],
                            agg.at[pl.ds(cid * N_NODES + r0, _RPT)])
            plsc.subcore_barrier()

    return sc_scatter2


_BF = jnp.bfloat16


def _dot(a, b):
    return jnp.dot(a, b, preferred_element_type=jnp.float32)


_WNAMES = ['sfw1', 'sfb1', 'sfw2', 'sfb2', 'w_bpnq_s', 'b_bpnq', 'w_bpnq_v',
           'w_f1s', 'b_f1', 'w_f1v', 's32', 's32t', 'w_f2s', 'b_f2', 'w_f2v',
           'w_g1', 'b_g1', 'w_g2', 'b_g2', 'wns', 'bns', 'wnv']


def _dense_body(e_in, esc, eattr, hl, hr, *wrefs, msgl, msgr, part):
    w = {k: wrefs[i][...] for i, k in enumerate(_WNAMES)}
    e = e_in[...].astype(_BF)
    v_b = e[:, NS:]
    x = _dot(jnp.concatenate([e[:, :NS], esc[...].astype(_BF)], axis=1),
             w['sfw1']) + w['sfb1']
    x = x * jax.nn.sigmoid(x)
    s_b = (_dot(x.astype(_BF), w['sfw2']) + w['sfb2']).astype(_BF)
    ea = eattr[...].astype(_BF)
    s_sh = ea[:, 0:1]
    v_sh = ea[:, 1:4]
    hlv = hl[...][:, :D].astype(_BF)
    hrv = hr[...][:, :D].astype(_BF)
    sblr = jnp.concatenate([s_b, hlv[:, :NS], hrv[:, :NS]], axis=1)
    vblr = jnp.concatenate([v_b, hlv[:, NS:], hrv[:, NS:]], axis=1)
    bpnq_s = _dot(sblr, w['w_bpnq_s']) + w['b_bpnq']
    bpnq_v = _dot(vblr, w['w_bpnq_v'])
    f_s = _dot(jnp.concatenate([bpnq_s.astype(_BF), s_sh], axis=1),
               w['w_f1s']) + w['b_f1']
    f_v = _dot(jnp.concatenate([bpnq_v.astype(_BF), v_sh], axis=1),
               w['w_f1v'])
    f_s = f_s * jax.nn.sigmoid(f_s)
    nrm = jnp.sqrt(_dot((f_v * f_v).astype(_BF), w['s32']))
    scale = jax.nn.sigmoid(nrm) * nrm / (nrm + 1e-8)
    f_v = f_v * _dot(scale.astype(_BF), w['s32t'])
    f_s = _dot(f_s.astype(_BF), w['w_f2s']) + w['b_f2']
    f_v = _dot(f_v.astype(_BF), w['w_f2v'])
    g = _dot(jnp.concatenate([sblr, s_sh], axis=1), w['w_g1']) + w['b_g1']
    g = g * jax.nn.sigmoid(g)
    gates = jax.nn.sigmoid(_dot(g.astype(_BF), w['w_g2']) + w['b_g2'])
    p_s = _dot(sblr, w['wns']) + w['bns']
    p_v = _dot(vblr, w['wnv'])
    msgl[...] = jnp.concatenate(
        [f_s[:, :NS], f_v[:, :NV]], axis=1) * gates[:, 0:1]
    msgr[...] = jnp.concatenate(
        [f_s[:, NS:], f_v[:, NV:]], axis=1) * gates[:, 1:2]
    part[...] = jnp.concatenate([p_s, p_v], axis=1)


_BE = 4000


def _dense_call(e_in, esc, eattr, hl, hr, weights):
    grid = (N_EDGES // _BE,)
    row_spec = lambda width: pl.BlockSpec((_BE, width), lambda i: (i, 0))
    full = lambda a: pl.BlockSpec(a.shape, lambda i: (0,) * a.ndim)
    in_specs = [row_spec(D), row_spec(16), row_spec(4), row_spec(DP),
                row_spec(DP)]
    in_specs += [full(a) for a in weights]
    out_specs = [row_spec(D)] * 3
    out_shape = [jax.ShapeDtypeStruct((N_EDGES, D), jnp.float32)] * 3

    def body(*refs):
        _dense_body(*refs[:-3], msgl=refs[-3], msgr=refs[-2], part=refs[-1])

    return pl.pallas_call(
        body, grid=grid, in_specs=in_specs, out_specs=out_specs,
        out_shape=out_shape,
        compiler_params=pltpu.CompilerParams(
            dimension_semantics=("arbitrary",)),
    )(e_in, esc, eattr, hl, hr, *weights)


def _combine_body(al, ar, ol, orr):
    zr = jnp.zeros((NP - N_NODES, D), jnp.float32)
    zl = jnp.zeros((NP, DP - D), jnp.float32)
    a = al[...]
    ol[...] = jnp.concatenate(
        [jnp.concatenate([a[:N_NODES] + a[N_NODES:], zr], axis=0), zl], axis=1)
    b = ar[...]
    orr[...] = jnp.concatenate(
        [jnp.concatenate([b[:N_NODES] + b[N_NODES:], zr], axis=0), zl], axis=1)


def _combine_call(agg_l2, agg_r2):
    return pl.pallas_call(
        _combine_body,
        out_shape=[jax.ShapeDtypeStruct((NP, DP), jnp.float32)] * 2,
    )(agg_l2, agg_r2)


_BA = 8000


def _final_add_body(p, fl, fr, o):
    o[...] = p[...] + fl[...][:, :D] + fr[...][:, :D]


def _final_add_call(part, fl, fr):
    spec = pl.BlockSpec((_BA, D), lambda i: (i, 0))
    pspec = pl.BlockSpec((_BA, DP), lambda i: (i, 0))
    return pl.pallas_call(
        _final_add_body, grid=(N_EDGES // _BA,),
        in_specs=[spec, pspec, pspec], out_specs=spec,
        out_shape=jax.ShapeDtypeStruct((N_EDGES, D), jnp.float32),
        compiler_params=pltpu.CompilerParams(
            dimension_semantics=("arbitrary",)),
    )(part, fl, fr)


def _kron3(w):
    return jnp.kron(w, jnp.eye(3, dtype=w.dtype))


def _expand_weights(p):
    r32, r16 = 1 / np.sqrt(32), 1 / np.sqrt(16)
    r33, r17 = 1 / np.sqrt(33), 1 / np.sqrt(17)
    r65 = 1 / np.sqrt(65)
    k3 = _kron3
    f32 = jnp.float32
    cat = jnp.concatenate

    w_bpnq_s = (jnp.zeros((96, 64), f32)
                .at[0:32, 0:16].set(p['L_bl_w0'] * r32)
                .at[0:32, 32:48].set(p['R_bl_w0'] * r32)
                .at[32:64, 16:32].set(p['L_nl_w0'] * r32)
                .at[64:96, 48:64].set(p['R_nl_w0'] * r32))
    b_bpnq = cat([p['L_bl_b'], p['L_nl_b'],
                  p['R_bl_b'], p['R_nl_b']])[None, :]
    w_bpnq_v = (jnp.zeros((144, 96), f32)
                .at[0:48, 0:24].set(k3(p['L_bl_w1']) * r16)
                .at[0:48, 48:72].set(k3(p['R_bl_w1']) * r16)
                .at[48:96, 24:48].set(k3(p['L_nl_w1']) * r16)
                .at[96:144, 72:96].set(k3(p['R_nl_w1']) * r16))
    w_f1s = (jnp.zeros((65, 64), f32)
             .at[0:32, 0:32].set(p['L_f1_w0'][0:32] * r33)
             .at[32:64, 32:64].set(p['R_f1_w0'][0:32] * r33)
             .at[64:65, 0:32].set(p['L_f1_w0'][32:33] * r33)
             .at[64:65, 32:64].set(p['R_f1_w0'][32:33] * r33))
    b_f1 = cat([p['L_f1_b'], p['R_f1_b']])[None, :]
    w_f1v = (jnp.zeros((99, 96), f32)
             .at[0:48, 0:48].set(k3(p['L_f1_w1'][0:16]) * r17)
             .at[48:96, 48:96].set(k3(p['R_f1_w1'][0:16]) * r17)
             .at[96:99, 0:48].set(k3(p['L_f1_w1'][16:17]) * r17)
             .at[96:99, 48:96].set(k3(p['R_f1_w1'][16:17]) * r17))
    s32 = jnp.kron(jnp.eye(32, dtype=f32), jnp.ones((3, 1), f32))
    w_f2s = (jnp.zeros((64, 64), f32)
             .at[0:32, 0:32].set(p['L_f2_w0'] * r32)
             .at[32:64, 32:64].set(p['R_f2_w0'] * r32))
    b_f2 = cat([p['L_f2_b'], p['R_f2_b']])[None, :]
    w_f2v = (jnp.zeros((96, 96), f32)
             .at[0:48, 0:48].set(k3(p['L_f2_w1']) * r16)
             .at[48:96, 48:96].set(k3(p['R_f2_w1']) * r16))
    w_g1 = (jnp.zeros((97, 130), f32)
            .at[0:32, 0:65].set(p['L_g1_w0'][0:32] * r65)
            .at[0:32, 65:130].set(p['R_g1_w0'][0:32] * r65)
            .at[32:64, 0:65].set(p['L_g1_w0'][32:64] * r65)
            .at[64:96, 65:130].set(p['R_g1_w0'][32:64] * r65)
            .at[96:97, 0:65].set(p['L_g1_w0'][64:65] * r65)
            .at[96:97, 65:130].set(p['R_g1_w0'][64:65] * r65))
    b_g1 = cat([p['L_g1_b'], p['R_g1_b']])[None, :]
    w_g2 = (jnp.zeros((130, 2), f32)
            .at[0:65, 0:1].set(p['L_g2_w0'] * r65)
            .at[65:130, 1:2].set(p['R_g2_w0'] * r65))
    b_g2 = cat([p['L_g2_b'], p['R_g2_b']])[None, :]
    wns = cat([p['self_w0'], p['nfl_w0'], p['nfr_w0']], axis=0) * r32
    bns = (p['nfl_b'] + p['nfr_b'] + p['self_b'])[None, :]
    wnv = cat([k3(p['self_w1']), k3(p['nfl_w1']), k3(p['nfr_w1'])],
              axis=0) * r16
    d = {
        'sfw1': p['sf_w1'], 'sfb1': p['sf_b1'][None, :],
        'sfw2': p['sf_w2'], 'sfb2': p['sf_b2'][None, :],
        'w_bpnq_s': w_bpnq_s, 'b_bpnq': b_bpnq, 'w_bpnq_v': w_bpnq_v,
        'w_f1s': w_f1s, 'b_f1': b_f1, 'w_f1v': w_f1v,
        's32': s32, 's32t': s32.T,
        'w_f2s': w_f2s, 'b_f2': b_f2, 'w_f2v': w_f2v,
        'w_g1': w_g1, 'b_g1': b_g1, 'w_g2': w_g2, 'b_g2': b_g2,
        'wns': wns, 'bns': bns, 'wnv': wnv,
    }
    out = []
    for k in _WNAMES:
        a = d[k]
        out.append(a if k.startswith('b') or k.startswith('sfb')
                   else a.astype(_BF))
    return out


def kernel(h, e_in, edge_scalars, edge_attr, edge_index, params):
    left = edge_index[0]
    right = edge_index[1]
    lp = jnp.pad(left, (0, EP - N_EDGES))
    rp = jnp.pad(right, (0, EP - N_EDGES))
    lg = lp.reshape(EP // _CG, _CG)
    rg = rp.reshape(EP // _CG, _CG)
    ls = left.reshape(N_EDGES // _CS, _CS)
    rs = right.reshape(N_EDGES // _CS, _CS)
    weights = _expand_weights(params)
    h_pad = jnp.pad(h, ((0, NP - N_NODES), (0, DP - D)))
    hl, hr = _get_gather2()(h_pad, lg, h_pad, rg)
    msgl, msgr, part = _dense_call(e_in, edge_scalars, edge_attr, hl, hr,
                                   weights)
    zeros_blk = jnp.zeros((_RPT, D), jnp.float32)
    agg_l2, agg_r2 = _get_scatter2()(msgl, rs, msgr, ls, zeros_blk)
    agg_l, agg_r = _combine_call(agg_l2, agg_r2)
    fl, fr = _get_gather2()(agg_l, lg, agg_r, rg)
    return _final_add_call(part, fl, fr)


# speedup vs baseline: 1.4506x; 1.3484x over previous
import functools

import jax
import jax.numpy as jnp
import numpy as np
from jax import lax
from jax.experimental import pallas as pl
from jax.experimental.pallas import tpu as pltpu
from jax.experimental.pallas import tpu_sc as plsc

N_NODES = 10000
N_EDGES = 160000
D = 80
NS = 32
NV = 48

_NC = 2
_NSUB = 16
_NW = _NC * _NSUB
_C = 500
_K = N_EDGES // (_C * _NW)
_RPT = N_NODES // _NSUB


@functools.lru_cache(maxsize=None)
def _get_gather1():
    mesh = plsc.VectorSubcoreMesh(core_axis_name="c", subcore_axis_name="s")

    @functools.partial(
        pl.kernel, mesh=mesh,
        out_type=jax.ShapeDtypeStruct((N_EDGES, D), jnp.float32),
        scratch_types=[pltpu.VMEM((_C,), jnp.int32),
                       pltpu.VMEM((_C,), jnp.int32),
                       pltpu.VMEM((_C, D), jnp.float32),
                       pltpu.VMEM((_C, D), jnp.float32),
                       pltpu.SemaphoreType.DMA,
                       pltpu.SemaphoreType.DMA],
        compiler_params=pltpu.CompilerParams(use_tc_tiling_on_sc=False),
    )
    def sc_gather1(tab, idx2, out, i0, i1, r0, r1, s0, s1):
        wid = lax.axis_index("s") * _NC + lax.axis_index("c")
        ibufs, rbufs, sems = (i0, i1), (r0, r1), (s0, s1)
        j0 = wid * _K
        pltpu.sync_copy(idx2.at[j0], ibufs[0])
        cp = pltpu.async_copy(tab.at[ibufs[0]], rbufs[0], sems[0])
        for k in range(_K):
            nb = (k + 1) % 2
            cpn = None
            if k + 1 < _K:
                pltpu.sync_copy(idx2.at[j0 + k + 1], ibufs[nb])
                cpn = pltpu.async_copy(tab.at[ibufs[nb]], rbufs[nb], sems[nb])
            cp.wait()
            pltpu.sync_copy(rbufs[k % 2], out.at[pl.ds((j0 + k) * _C, _C)])
            cp = cpn

    return sc_gather1


@functools.lru_cache(maxsize=None)
def _get_scatter1():
    mesh = plsc.VectorSubcoreMesh(core_axis_name="c", subcore_axis_name="s")

    @functools.partial(
        pl.kernel, mesh=mesh,
        out_type=jax.ShapeDtypeStruct((_NC * N_NODES, D), jnp.float32),
        scratch_types=[pltpu.VMEM((_C,), jnp.int32),
                       pltpu.VMEM((_C,), jnp.int32),
                       pltpu.VMEM((_C, D), jnp.float32),
                       pltpu.VMEM((_C, D), jnp.float32),
                       pltpu.VMEM_SHARED((N_NODES, D), jnp.float32),
                       pltpu.SemaphoreType.DMA,
                       pltpu.SemaphoreType.DMA],
        compiler_params=pltpu.CompilerParams(use_tc_tiling_on_sc=False),
    )
    def sc_scatter1(msg, idx2, zeros_blk, agg, i0, i1, m0, m1, acc, s0, s1):
        cid = lax.axis_index("c")
        sid = lax.axis_index("s")
        wid = sid * _NC + cid
        r0 = sid * _RPT
        ibufs, mbufs, sems = (i0, i1), (m0, m1), (s0, s1)
        pltpu.sync_copy(zeros_blk, acc.at[pl.ds(r0, _RPT)])
        plsc.subcore_barrier()
        j0 = wid * _K
        pltpu.sync_copy(idx2.at[j0], ibufs[0])
        cp = pltpu.async_copy(msg.at[pl.ds(j0 * _C, _C)], mbufs[0], sems[0])
        for k in range(_K):
            nb = (k + 1) % 2
            cpn = None
            if k + 1 < _K:
                pltpu.sync_copy(idx2.at[j0 + k + 1], ibufs[nb])
                cpn = pltpu.async_copy(msg.at[pl.ds((j0 + k + 1) * _C, _C)],
                                       mbufs[nb], sems[nb])
            cp.wait()
            pltpu.sync_copy(mbufs[k % 2], acc.at[ibufs[k % 2]], add=True)
            cp = cpn
        plsc.subcore_barrier()
        pltpu.sync_copy(acc.at[pl.ds(r0, _RPT)],
                        agg.at[pl.ds(cid * N_NODES + r0, _RPT)])

    return sc_scatter1


_BF = jnp.bfloat16


def _dot(a, b):
    return jnp.dot(a, b, preferred_element_type=jnp.float32)


_WNAMES = ['sfw1', 'sfb1', 'sfw2', 'sfb2', 'w_bpnq_s', 'b_bpnq', 'w_bpnq_v',
           'w_f1s', 'b_f1', 'w_f1v', 's32', 's32t', 'w_f2s', 'b_f2', 'w_f2v',
           'w_g1', 'b_g1', 'w_g2', 'b_g2', 'wns', 'bns', 'wnv']


def _dense_body(e_in, esc, eattr, hl, hr, *wrefs, msgl, msgr, part):
    w = {k: wrefs[i][...] for i, k in enumerate(_WNAMES)}
    e = e_in[...].astype(_BF)
    v_b = e[:, NS:]
    x = _dot(jnp.concatenate([e[:, :NS], esc[...].astype(_BF)], axis=1),
             w['sfw1']) + w['sfb1']
    x = x * jax.nn.sigmoid(x)
    s_b = (_dot(x.astype(_BF), w['sfw2']) + w['sfb2']).astype(_BF)
    ea = eattr[...].astype(_BF)
    s_sh = ea[:, 0:1]
    v_sh = ea[:, 1:4]
    hlv = hl[...].astype(_BF)
    hrv = hr[...].astype(_BF)
    sblr = jnp.concatenate([s_b, hlv[:, :NS], hrv[:, :NS]], axis=1)
    vblr = jnp.concatenate([v_b, hlv[:, NS:], hrv[:, NS:]], axis=1)
    bpnq_s = _dot(sblr, w['w_bpnq_s']) + w['b_bpnq']
    bpnq_v = _dot(vblr, w['w_bpnq_v'])
    f_s = _dot(jnp.concatenate([bpnq_s.astype(_BF), s_sh], axis=1),
               w['w_f1s']) + w['b_f1']
    f_v = _dot(jnp.concatenate([bpnq_v.astype(_BF), v_sh], axis=1),
               w['w_f1v'])
    f_s = f_s * jax.nn.sigmoid(f_s)
    nrm = jnp.sqrt(_dot((f_v * f_v).astype(_BF), w['s32']))
    scale = jax.nn.sigmoid(nrm) * nrm / (nrm + 1e-8)
    f_v = f_v * _dot(scale.astype(_BF), w['s32t'])
    f_s = _dot(f_s.astype(_BF), w['w_f2s']) + w['b_f2']
    f_v = _dot(f_v.astype(_BF), w['w_f2v'])
    g = _dot(jnp.concatenate([sblr, s_sh], axis=1), w['w_g1']) + w['b_g1']
    g = g * jax.nn.sigmoid(g)
    gates = jax.nn.sigmoid(_dot(g.astype(_BF), w['w_g2']) + w['b_g2'])
    p_s = _dot(sblr, w['wns']) + w['bns']
    p_v = _dot(vblr, w['wnv'])
    msgl[...] = jnp.concatenate(
        [f_s[:, :NS], f_v[:, :NV]], axis=1) * gates[:, 0:1]
    msgr[...] = jnp.concatenate(
        [f_s[:, NS:], f_v[:, NV:]], axis=1) * gates[:, 1:2]
    part[...] = jnp.concatenate([p_s, p_v], axis=1)


_BE = 4000


def _dense_call(e_in, esc, eattr, hl, hr, weights):
    grid = (N_EDGES // _BE,)
    row_spec = lambda width: pl.BlockSpec((_BE, width), lambda i: (i, 0))
    full = lambda a: pl.BlockSpec(a.shape, lambda i: (0,) * a.ndim)
    in_specs = [row_spec(D), row_spec(16), row_spec(4), row_spec(D),
                row_spec(D)]
    in_specs += [full(a) for a in weights]
    out_specs = [row_spec(D)] * 3
    out_shape = [jax.ShapeDtypeStruct((N_EDGES, D), jnp.float32)] * 3

    def body(*refs):
        _dense_body(*refs[:-3], msgl=refs[-3], msgr=refs[-2], part=refs[-1])

    return pl.pallas_call(
        body, grid=grid, in_specs=in_specs, out_specs=out_specs,
        out_shape=out_shape,
        compiler_params=pltpu.CompilerParams(
            dimension_semantics=("arbitrary",)),
    )(e_in, esc, eattr, hl, hr, *weights)


def _combine_body(al, ar, ol, orr):
    a = al[...]
    ol[...] = a[:N_NODES] + a[N_NODES:]
    b = ar[...]
    orr[...] = b[:N_NODES] + b[N_NODES:]


def _combine_call(agg_l2, agg_r2):
    return pl.pallas_call(
        _combine_body,
        out_shape=[jax.ShapeDtypeStruct((N_NODES, D), jnp.float32)] * 2,
    )(agg_l2, agg_r2)


_BA = 8000


def _final_add_body(p, fl, fr, o):
    o[...] = p[...] + fl[...] + fr[...]


def _final_add_call(part, fl, fr):
    spec = pl.BlockSpec((_BA, D), lambda i: (i, 0))
    return pl.pallas_call(
        _final_add_body, grid=(N_EDGES // _BA,),
        in_specs=[spec] * 3, out_specs=spec,
        out_shape=jax.ShapeDtypeStruct((N_EDGES, D), jnp.float32),
        compiler_params=pltpu.CompilerParams(
            dimension_semantics=("arbitrary",)),
    )(part, fl, fr)


def _kron3(w):
    return jnp.kron(w, jnp.eye(3, dtype=w.dtype))


def _expand_weights(p):
    r32, r16 = 1 / np.sqrt(32), 1 / np.sqrt(16)
    r33, r17 = 1 / np.sqrt(33), 1 / np.sqrt(17)
    r65 = 1 / np.sqrt(65)
    k3 = _kron3
    f32 = jnp.float32
    cat = jnp.concatenate

    w_bpnq_s = (jnp.zeros((96, 64), f32)
                .at[0:32, 0:16].set(p['L_bl_w0'] * r32)
                .at[0:32, 32:48].set(p['R_bl_w0'] * r32)
                .at[32:64, 16:32].set(p['L_nl_w0'] * r32)
                .at[64:96, 48:64].set(p['R_nl_w0'] * r32))
    b_bpnq = cat([p['L_bl_b'], p['L_nl_b'],
                  p['R_bl_b'], p['R_nl_b']])[None, :]
    w_bpnq_v = (jnp.zeros((144, 96), f32)
                .at[0:48, 0:24].set(k3(p['L_bl_w1']) * r16)
                .at[0:48, 48:72].set(k3(p['R_bl_w1']) * r16)
                .at[48:96, 24:48].set(k3(p['L_nl_w1']) * r16)
                .at[96:144, 72:96].set(k3(p['R_nl_w1']) * r16))
    w_f1s = (jnp.zeros((65, 64), f32)
             .at[0:32, 0:32].set(p['L_f1_w0'][0:32] * r33)
             .at[32:64, 32:64].set(p['R_f1_w0'][0:32] * r33)
             .at[64:65, 0:32].set(p['L_f1_w0'][32:33] * r33)
             .at[64:65, 32:64].set(p['R_f1_w0'][32:33] * r33))
    b_f1 = cat([p['L_f1_b'], p['R_f1_b']])[None, :]
    w_f1v = (jnp.zeros((99, 96), f32)
             .at[0:48, 0:48].set(k3(p['L_f1_w1'][0:16]) * r17)
             .at[48:96, 48:96].set(k3(p['R_f1_w1'][0:16]) * r17)
             .at[96:99, 0:48].set(k3(p['L_f1_w1'][16:17]) * r17)
             .at[96:99, 48:96].set(k3(p['R_f1_w1'][16:17]) * r17))
    s32 = jnp.kron(jnp.eye(32, dtype=f32), jnp.ones((3, 1), f32))
    w_f2s = (jnp.zeros((64, 64), f32)
             .at[0:32, 0:32].set(p['L_f2_w0'] * r32)
             .at[32:64, 32:64].set(p['R_f2_w0'] * r32))
    b_f2 = cat([p['L_f2_b'], p['R_f2_b']])[None, :]
    w_f2v = (jnp.zeros((96, 96), f32)
             .at[0:48, 0:48].set(k3(p['L_f2_w1']) * r16)
             .at[48:96, 48:96].set(k3(p['R_f2_w1']) * r16))
    w_g1 = (jnp.zeros((97, 130), f32)
            .at[0:32, 0:65].set(p['L_g1_w0'][0:32] * r65)
            .at[0:32, 65:130].set(p['R_g1_w0'][0:32] * r65)
            .at[32:64, 0:65].set(p['L_g1_w0'][32:64] * r65)
            .at[64:96, 65:130].set(p['R_g1_w0'][32:64] * r65)
            .at[96:97, 0:65].set(p['L_g1_w0'][64:65] * r65)
            .at[96:97, 65:130].set(p['R_g1_w0'][64:65] * r65))
    b_g1 = cat([p['L_g1_b'], p['R_g1_b']])[None, :]
    w_g2 = (jnp.zeros((130, 2), f32)
            .at[0:65, 0:1].set(p['L_g2_w0'] * r65)
            .at[65:130, 1:2].set(p['R_g2_w0'] * r65))
    b_g2 = cat([p['L_g2_b'], p['R_g2_b']])[None, :]
    wns = cat([p['self_w0'], p['nfl_w0'], p['nfr_w0']], axis=0) * r32
    bns = (p['nfl_b'] + p['nfr_b'] + p['self_b'])[None, :]
    wnv = cat([k3(p['self_w1']), k3(p['nfl_w1']), k3(p['nfr_w1'])],
              axis=0) * r16
    d = {
        'sfw1': p['sf_w1'], 'sfb1': p['sf_b1'][None, :],
        'sfw2': p['sf_w2'], 'sfb2': p['sf_b2'][None, :],
        'w_bpnq_s': w_bpnq_s, 'b_bpnq': b_bpnq, 'w_bpnq_v': w_bpnq_v,
        'w_f1s': w_f1s, 'b_f1': b_f1, 'w_f1v': w_f1v,
        's32': s32, 's32t': s32.T,
        'w_f2s': w_f2s, 'b_f2': b_f2, 'w_f2v': w_f2v,
        'w_g1': w_g1, 'b_g1': b_g1, 'w_g2': w_g2, 'b_g2': b_g2,
        'wns': wns, 'bns': bns, 'wnv': wnv,
    }
    out = []
    for k in _WNAMES:
        a = d[k]
        out.append(a if k.startswith('b') or k.startswith('sfb')
                   else a.astype(_BF))
    return out


def kernel(h, e_in, edge_scalars, edge_attr, edge_index, params):
    left = edge_index[0]
    right = edge_index[1]
    lg = left.reshape(N_EDGES // _C, _C)
    rg = right.reshape(N_EDGES // _C, _C)
    weights = _expand_weights(params)
    g1 = _get_gather1()
    s1 = _get_scatter1()
    hl = g1(h, lg)
    hr = g1(h, rg)
    msgl, msgr, part = _dense_call(e_in, edge_scalars, edge_attr, hl, hr,
                                   weights)
    zeros_blk = jnp.zeros((_RPT, D), jnp.float32)
    agg_l2 = s1(msgl, rg, zeros_blk)
    agg_r2 = s1(msgr, lg, zeros_blk)
    agg_l, agg_r = _combine_call(agg_l2, agg_r2)
    fl = g1(agg_l, lg)
    fr = g1(agg_r, rg)
    return _final_add_call(part, fl, fr)
